# baseline (speedup 1.0000x reference)
import functools

import jax
import jax.numpy as jnp
from jax import lax
from jax.experimental import pallas as pl
from jax.experimental.pallas import tpu as pltpu
from jax.experimental.pallas import tpu_sc as plsc

NC = 2
NS = 16
L = 16
CHUNK = 128
W = 128
NWIN = 7813
NSLOT = 8
FLUSH = 32
QCAP = 2048


def _extract_body(batch, ut_hbm, it_hbm, usr_hbm, itm_hbm, ug_hbm, ig_hbm,
                  idxv, hitsv, hposv, wslots, colbuf, posbuf, mqu, mqp,
                  sem0, sem1, sem2, sem3, sem4, sem5, sem6, sem7):
  sems = (sem0, sem1, sem2, sem3, sem4, sem5, sem6, sem7)
  wid = lax.axis_index("s") * NC + lax.axis_index("c")
  nw = NC * NS
  wper = NWIN // nw
  wext = NWIN - wper * nw
  wlo = wid * wper + jnp.minimum(wid, wext)
  nwin = wper + (wid < wext).astype(jnp.int32)
  MB = NSLOT // 2
  n_macro = (wper + 1 + MB - 1) // MB

  lane = lax.iota(jnp.int32, L)
  ihalf = batch // 2

  for side_hbm, tab_hbm, g_hbm in ((usr_hbm, ut_hbm, ug_hbm),
                                   (itm_hbm, it_hbm, ig_hbm)):
    def compact(t, cnt):
      v = idxv[pl.ds((t % (ihalf // L)) * L, L)]
      w = jax.lax.shift_right_logical(v, 7)
      m = (w >= wlo) & (w < wlo + nwin)
      plsc.store_compressed(hitsv.at[pl.ds(cnt, L)], v, mask=m)
      plsc.store_compressed(hposv.at[pl.ds(cnt, L)], t * L + lane, mask=m)
      return cnt + plsc.all_reduce_population_count(m)[0]

    cnt = jnp.int32(0)
    for h in range(2):
      pltpu.sync_copy(side_hbm.at[pl.ds(h * ihalf, ihalf)], idxv)
      cnt = lax.fori_loop(h * (ihalf // L), (h + 1) * (ihalf // L),
                          compact, cnt)
    hitsv[pl.ds(cnt, L)] = jnp.full((L,), -1, jnp.int32)
    nhv = (cnt + L - 1) // L

    def fire(k, slot):
      wk = wlo + jnp.minimum(k, nwin - 1)
      ua = pl.multiple_of(wk * W, W)
      pltpu.async_copy(tab_hbm.at[:, pl.ds(ua, W)], wslots.at[slot],
                       sems[slot])

    def drain(slot):
      pltpu.make_async_copy(tab_hbm.at[:, pl.ds(0, W)], wslots.at[slot],
                            sems[slot]).wait()

    def flush_batches(qc):
      def fl(fi, carry):
        for t in range(FLUSH // L):
          h0 = fi * FLUSH + t * L
          hvec = mqu[pl.ds(h0, L)]
          pvec = mqp[pl.ds(h0, L)]
          valid = (h0 + lane) < qc
          slotv = (jax.lax.shift_right_logical(hvec, 7) - wlo) & (NSLOT - 1)
          colv = hvec & (W - 1)
          posbuf[0, pl.ds(t * L, L)] = jnp.where(
              valid, pvec, jnp.full((L,), batch, jnp.int32))
          rowv = t * L + lane
          for f in range(64):
            fsp = jnp.full((L,), f, jnp.int32)
            val = plsc.load_gather(wslots, [slotv, fsp, colv])
            plsc.store_scatter(colbuf, [rowv, fsp], val, mask=valid)
        pltpu.sync_copy(colbuf, g_hbm.at[posbuf.at[0]])
        return carry
      lax.fori_loop(0, (qc + FLUSH - 1) // FLUSH, fl, 0)

    for s in range(MB):
      fire(s, s)

    def macro_pair(p, carry):
      for half in range(2):
        mi = p * 2 + half
        k4 = mi * MB
        sbase = (half * MB) % NSLOT
        for s in range(MB):
          fire(k4 + MB + s, (sbase + MB + s) % NSLOT)
        for s in range(MB):
          drain(sbase + s)

        def scan_vreg(t, qc):
          hv = hitsv[pl.ds(t * L, L)]
          pv = hposv[pl.ds(t * L, L)]
          d = jax.lax.shift_right_logical(hv, 7) - wlo
          m = (d >= k4) & (d < k4 + MB) & (d < nwin)
          plsc.store_compressed(mqu.at[pl.ds(qc, L)], hv, mask=m)
          plsc.store_compressed(mqp.at[pl.ds(qc, L)], pv, mask=m)
          qc = qc + plsc.all_reduce_population_count(m)[0]

          @pl.when(qc >= QCAP - 2 * L)
          def _():
            flush_batches(qc)

          return jnp.where(qc >= QCAP - 2 * L, 0, qc)

        qc = lax.fori_loop(0, nhv, scan_vreg, jnp.int32(0))

        @pl.when(qc > 0)
        def _():
          flush_batches(qc)

      return carry

    lax.fori_loop(0, n_macro // 2, macro_pair, 0)
    for s in range(MB):
      drain(s)


def _dot_body(batch, b_per_w, ug_hbm, ig_hbm, ub_hbm, ib_hbm, usr_hbm,
              itm_hbm, out_hbm, usr_v, itm_v, urows, irows, ubv, ibv,
              outv, bsem, csem):
  wid = lax.axis_index("s") * NC + lax.axis_index("c")
  base = wid * b_per_w

  pltpu.sync_copy(usr_hbm.at[pl.ds(base, b_per_w)], usr_v)
  pltpu.sync_copy(itm_hbm.at[pl.ds(base, b_per_w)], itm_v)
  bias_copies = []
  for g in range(b_per_w // CHUNK):
    sl = pl.ds(g * CHUNK, CHUNK)
    bias_copies.append(
        pltpu.async_copy(ub_hbm.at[usr_v.at[sl]], ubv.at[sl], bsem))
    bias_copies.append(
        pltpu.async_copy(ib_hbm.at[itm_v.at[sl]], ibv.at[sl], bsem))
  for c in bias_copies:
    c.wait()

  lane = lax.iota(jnp.int32, L)
  dnums = lax.GatherDimensionNumbers(
      offset_dims=(), collapsed_slice_dims=(0,), start_index_map=(0,))

  def shufxor(x, k):
    return lax.gather(x, (lane ^ k)[:, None], dnums, (1,),
                      mode=lax.GatherScatterMode.PROMISE_IN_BOUNDS)

  GR = 128
  gper = GR // L

  def group(g, carry):
    @pl.when(g % gper == 0)
    def _():
      ch = g // gper
      rsl = pl.ds(base + ch * GR, GR)
      pltpu.sync_copy(ug_hbm.at[rsl], urows)
      pltpu.sync_copy(ig_hbm.at[rsl], irows)

    vecs = []
    for j in range(L):
      r = (g % gper) * L + j
      acc = urows[r, pl.ds(0, L)] * irows[r, pl.ds(0, L)]
      for c in range(1, 4):
        acc = acc + urows[r, pl.ds(c * L, L)] * irows[r, pl.ds(c * L, L)]
      vecs.append(acc)
    for k in (1, 2, 4, 8):
      nxt = []
      sel = (lane & k) == 0
      for p in range(0, len(vecs), 2):
        a, b = vecs[p], vecs[p + 1]
        nxt.append(jnp.where(sel, a + shufxor(a, k), b + shufxor(b, k)))
      vecs = nxt
    sl = pl.ds(g * L, L)
    outv[sl] = vecs[0] + ubv[sl] + ibv[sl]
    return carry

  lax.fori_loop(0, b_per_w // L, group, 0)
  pltpu.sync_copy(outv, out_hbm.at[pl.ds(base, b_per_w)])


def kernel(uEmbeds, iEmbeds, uBias, iBias, usr, itm):
  batch = usr.shape[0]
  latdim = uEmbeds.shape[1]
  nw = NC * NS
  b_per_w = batch // nw
  uT = uEmbeds.T
  iT = iEmbeds.T
  mesh = plsc.VectorSubcoreMesh(
      core_axis_name="c", subcore_axis_name="s", num_cores=NC,
      num_subcores=NS)
  params = pltpu.CompilerParams(
      use_tc_tiling_on_sc=True, needs_layout_passes=False)

  extract = pl.kernel(
      functools.partial(_extract_body, batch),
      out_type=(
          jax.ShapeDtypeStruct((batch + FLUSH, W), jnp.float32),
          jax.ShapeDtypeStruct((batch + FLUSH, W), jnp.float32),
      ),
      mesh=mesh,
      scratch_types=[
          pltpu.VMEM((batch // 2,), jnp.int32),
          pltpu.VMEM((batch + L,), jnp.int32),
          pltpu.VMEM((batch + L,), jnp.int32),
          pltpu.VMEM((NSLOT, latdim, W), jnp.float32),
          pltpu.VMEM((FLUSH, W), jnp.float32),
          pltpu.VMEM((1, FLUSH), jnp.int32),
          pltpu.VMEM((QCAP + L,), jnp.int32),
          pltpu.VMEM((QCAP + L,), jnp.int32),
          pltpu.SemaphoreType.DMA,
          pltpu.SemaphoreType.DMA,
          pltpu.SemaphoreType.DMA,
          pltpu.SemaphoreType.DMA,
          pltpu.SemaphoreType.DMA,
          pltpu.SemaphoreType.DMA,
          pltpu.SemaphoreType.DMA,
          pltpu.SemaphoreType.DMA,
      ],
      compiler_params=params,
  )
  uG, iG = extract(uT, iT, usr, itm)

  dot = pl.kernel(
      functools.partial(_dot_body, batch, b_per_w),
      out_type=jax.ShapeDtypeStruct((batch,), jnp.float32),
      mesh=mesh,
      scratch_types=[
          pltpu.VMEM((b_per_w,), jnp.int32),
          pltpu.VMEM((b_per_w,), jnp.int32),
          pltpu.VMEM((128, W), jnp.float32),
          pltpu.VMEM((128, W), jnp.float32),
          pltpu.VMEM((b_per_w,), jnp.float32),
          pltpu.VMEM((b_per_w,), jnp.float32),
          pltpu.VMEM((b_per_w,), jnp.float32),
          pltpu.SemaphoreType.DMA,
          pltpu.SemaphoreType.DMA,
      ],
      compiler_params=params,
  )
  return dot(uG, iG, uBias, iBias, usr, itm)
